# baseline (speedup 1.0000x reference)
import functools

import jax
import jax.numpy as jnp
from jax import lax
from jax.experimental import pallas as pl
from jax.experimental.pallas import tpu as pltpu
from jax.experimental.pallas import tpu_sc as plsc

N_USERS = 5000
N_ITEMS = 5000
N_NODES = N_USERS + N_ITEMS
EMB = 128
E = 320000
BATCH = 4096
TAU = 0.2
SSL_LAMBDA = 0.1
REG_LAMBDA = 1e-4

NC = 2
NS = 16
NW = NC * NS
LANES = 16
G = EMB // LANES

CHUNK = 96
CHUNKS = 106
E_PAD = NW * CHUNKS * CHUNK
ROWS_PER_TILE = 624
REM_BASE = NS * ROWS_PER_TILE
REM = N_NODES - REM_BASE

_mesh = plsc.VectorSubcoreMesh(core_axis_name="c", subcore_axis_name="s")


@functools.partial(
    pl.kernel,
    mesh=_mesh,
    out_type=jax.ShapeDtypeStruct((NC, N_NODES, EMB), jnp.float32),
    scratch_types=[
        pltpu.VMEM((2, 2, CHUNK), jnp.int32),
        pltpu.VMEM((2, CHUNK, LANES), jnp.float32),
        pltpu.VMEM((2, CHUNK, EMB), jnp.float32),
        pltpu.VMEM_SHARED((N_NODES, EMB), jnp.float32),
        pltpu.SemaphoreType.DMA,
        pltpu.SemaphoreType.DMA,
        pltpu.SemaphoreType.DMA,
        pltpu.SemaphoreType.DMA,
    ],
)
def _prop(emb_hbm, idx_hbm, vals_hbm, out_hbm,
          idx_v, vals_v, buf, acc, six0, six1, sgv0, sgv1):
    cid = lax.axis_index("c")
    sid = lax.axis_index("s")
    wid = sid * NC + cid
    six = (six0, six1)
    sgv = (sgv0, sgv1)

    def fire_idx(j, s):
        pltpu.async_copy(idx_hbm.at[wid, j], idx_v.at[s], six[s])

    def wait_idx(j, s):
        pltpu.make_async_copy(idx_hbm.at[wid, j], idx_v.at[s], six[s]).wait()

    def fire_gv(j, s):
        pltpu.async_copy(emb_hbm.at[idx_v.at[s, 0]], buf.at[s], sgv[s])
        pltpu.async_copy(vals_hbm.at[wid, j], vals_v.at[s], sgv[s])

    def wait_gv(j, s):
        pltpu.make_async_copy(emb_hbm.at[idx_v.at[s, 0]], buf.at[s], sgv[s]).wait()
        pltpu.make_async_copy(vals_hbm.at[wid, j], vals_v.at[s], sgv[s]).wait()

    def scale_scatter(s):
        def _scale(i, c2):
            r0 = i * 4
            for dr in range(4):
                v = vals_v[s, r0 + dr]
                for g in range(G):
                    sl = pl.ds(g * LANES, LANES)
                    buf[s, r0 + dr, sl] = buf[s, r0 + dr, sl] * v
            return c2
        lax.fori_loop(0, CHUNK // 4, _scale, 0)
        pltpu.sync_copy(buf.at[s], acc.at[idx_v.at[s, 1]], add=True)

    def step(j, s, prefetch=True, fetch_idx=True):
        wait_gv(j, s)
        if prefetch:
            wait_idx(j + 1, 1 - s)
            fire_gv(j + 1, 1 - s)
        scale_scatter(s)
        if fetch_idx:
            fire_idx(j + 2, s)

    def _zrow(r, c):
        for g in range(G):
            buf[0, r, pl.ds(g * LANES, LANES)] = jnp.zeros((LANES,), jnp.float32)
        return c
    lax.fori_loop(0, CHUNK, _zrow, 0)
    base = sid * ROWS_PER_TILE
    for off in range(0, ROWS_PER_TILE, CHUNK):
        ln = min(CHUNK, ROWS_PER_TILE - off)
        pltpu.sync_copy(buf.at[0, pl.ds(0, ln)], acc.at[pl.ds(base + off, ln)])

    @pl.when(sid == NS - 1)
    def _():
        pltpu.sync_copy(buf.at[0, pl.ds(0, REM)], acc.at[pl.ds(REM_BASE, REM)])
    plsc.subcore_barrier()

    fire_idx(0, 0)
    wait_idx(0, 0)
    fire_gv(0, 0)
    fire_idx(1, 1)
    step(0, 0)

    def _pair(jj, c):
        step(2 * jj + 1, 1)
        step(2 * jj + 2, 0)
        return c
    lax.fori_loop(0, (CHUNKS - 4) // 2, _pair, 0)

    step(CHUNKS - 3, 1)
    step(CHUNKS - 2, 0, fetch_idx=False)
    step(CHUNKS - 1, 1, prefetch=False, fetch_idx=False)

    plsc.subcore_barrier()
    for off in range(0, ROWS_PER_TILE, CHUNK):
        ln = min(CHUNK, ROWS_PER_TILE - off)
        pltpu.sync_copy(acc.at[pl.ds(base + off, ln)],
                        out_hbm.at[cid, pl.ds(base + off, ln)])

    @pl.when(sid == NS - 1)
    def _():
        pltpu.sync_copy(acc.at[pl.ds(REM_BASE, REM)],
                        out_hbm.at[cid, pl.ds(REM_BASE, REM)])


@functools.partial(
    pl.kernel,
    mesh=_mesh,
    out_type=jax.ShapeDtypeStruct((3, BATCH, EMB), jnp.float32),
    scratch_types=[
        pltpu.VMEM((1, BATCH // NW), jnp.int32),
        pltpu.VMEM((BATCH // NW, EMB), jnp.float32),
        pltpu.SemaphoreType.DMA,
    ],
)
def _bgather(final_hbm, idx_hbm, out_hbm, idx_v, buf, sem):
    cid = lax.axis_index("c")
    sid = lax.axis_index("s")
    wid = sid * NC + cid
    bch = BATCH // NW
    base = wid * bch
    for t in range(3):
        pltpu.sync_copy(idx_hbm.at[pl.ds(t, 1), pl.ds(base, bch)], idx_v)
        if t > 0:
            for g in range(bch // LANES):
                sl = pl.ds(g * LANES, LANES)
                idx_v[0, sl] = idx_v[0, sl] + N_USERS
        pltpu.async_copy(final_hbm.at[idx_v.at[0]], buf, sem).wait()
        pltpu.sync_copy(buf, out_hbm.at[t, pl.ds(base, bch)])


def _combine_body(p_ref, s_ref, e_ref, so_ref, *, scale):
    e = p_ref[0] + p_ref[1]
    e_ref[...] = e
    so_ref[...] = (s_ref[...] + e) * scale


def _combine(p, s, scale):
    return pl.pallas_call(
        functools.partial(_combine_body, scale=scale),
        out_shape=(jax.ShapeDtypeStruct((N_NODES, EMB), jnp.float32),
                   jax.ShapeDtypeStruct((N_NODES, EMB), jnp.float32)),
    )(p, s)


_LB = 512
_LGRID = BATCH // _LB


def _loss_body(u_ref, pos_ref, neg_ref, out_ref, pn_ref, acc_ref):
    i = pl.program_id(0)

    @pl.when(i == 0)
    def _():
        pf = pos_ref[...]
        nrm = jnp.sqrt(jnp.sum(pf * pf, axis=1, keepdims=True))
        pn_ref[...] = pf / (nrm + 1e-12)

    sl = pl.ds(i * _LB, _LB)
    u = u_ref[sl, :]
    p = pos_ref[sl, :]
    ng = neg_ref[sl, :]

    pos_s = jnp.sum(u * p, axis=1)
    neg_s = jnp.sum(u * ng, axis=1)
    x = pos_s - neg_s
    log_sig = jnp.minimum(x, 0.0) - jnp.log1p(jnp.exp(-jnp.abs(x)))
    bpr_part = jnp.sum(log_sig)

    reg_part = jnp.sum(u * u) + jnp.sum(p * p) + jnp.sum(ng * ng)

    un = u / (jnp.sqrt(jnp.sum(u * u, axis=1, keepdims=True)) + 1e-12)
    pn_b = pn_ref[sl, :]
    pos_score = jnp.sum(un * pn_b, axis=1) / TAU

    logits = lax.dot_general(un, pn_ref[...], (((1,), (1,)), ((), ())),
                             preferred_element_type=jnp.float32) / TAU
    m = jnp.max(logits, axis=1)
    ttl = m + jnp.log(jnp.sum(jnp.exp(logits - m[:, None]), axis=1))
    na_part = jnp.sum(ttl - pos_score)

    @pl.when(i == 0)
    def _():
        acc_ref[0] = bpr_part
        acc_ref[1] = reg_part
        acc_ref[2] = na_part

    @pl.when(i > 0)
    def _():
        acc_ref[0] += bpr_part
        acc_ref[1] += reg_part
        acc_ref[2] += na_part

    @pl.when(i == _LGRID - 1)
    def _():
        bpr = -acc_ref[0] / BATCH
        reg = 0.5 * acc_ref[1] / BATCH
        na = acc_ref[2] / BATCH
        out_ref[0] = bpr + REG_LAMBDA * reg + SSL_LAMBDA * na


def _loss(u, pos, neg):
    full = pl.BlockSpec((BATCH, EMB), lambda i: (0, 0))
    return pl.pallas_call(
        _loss_body,
        grid=(_LGRID,),
        in_specs=[full, full, full],
        out_specs=pl.BlockSpec(memory_space=pltpu.SMEM),
        out_shape=jax.ShapeDtypeStruct((1,), jnp.float32),
        scratch_shapes=[
            pltpu.VMEM((BATCH, EMB), jnp.float32),
            pltpu.SMEM((3,), jnp.float32),
        ],
    )(u, pos, neg)


def kernel(user, pos_item, neg_item, adj_indices, adj_values,
           user_emb_w, item_emb_w):
    emb0 = jnp.concatenate([user_emb_w, item_emb_w], axis=0)

    rows = adj_indices[0].astype(jnp.int32)
    cols = adj_indices[1].astype(jnp.int32)
    vals = adj_values.astype(jnp.float32)
    pad = E_PAD - E
    rows_p = jnp.pad(rows, (0, pad)).reshape(NW, CHUNKS, CHUNK)
    cols_p = jnp.pad(cols, (0, pad)).reshape(NW, CHUNKS, CHUNK)
    idx_pack = jnp.stack([cols_p, rows_p], axis=2)
    vals16 = jnp.broadcast_to(jnp.pad(vals, (0, pad))[:, None],
                              (E_PAD, LANES)).reshape(NW, CHUNKS, CHUNK, LANES)

    e = emb0
    s = emb0
    for layer in range(3):
        parts = _prop(e, idx_pack, vals16)
        scale = 0.25 if layer == 2 else 1.0
        e, s = _combine(parts, s, scale)
    final = s

    idx = jnp.stack([user.astype(jnp.int32),
                     pos_item.astype(jnp.int32),
                     neg_item.astype(jnp.int32)])
    upn = _bgather(final, idx)
    total = _loss(upn[0], upn[1], upn[2])
    return total.reshape(())


# speedup vs baseline: 1.1698x; 1.1698x over previous
import functools

import jax
import jax.numpy as jnp
from jax import lax
from jax.experimental import pallas as pl
from jax.experimental.pallas import tpu as pltpu
from jax.experimental.pallas import tpu_sc as plsc

N_USERS = 5000
N_ITEMS = 5000
N_NODES = N_USERS + N_ITEMS
EMB = 128
E = 320000
BATCH = 4096
TAU = 0.2
SSL_LAMBDA = 0.1
REG_LAMBDA = 1e-4

NC = 2
NS = 16
NW = NC * NS
LANES = 16
G = EMB // LANES

CHUNK = 32
CHUNKS = 314
E_PAD = NW * CHUNKS * CHUNK
ROWS_PER_TILE = 624
REM_BASE = NS * ROWS_PER_TILE
REM = N_NODES - REM_BASE

_mesh = plsc.VectorSubcoreMesh(core_axis_name="c", subcore_axis_name="s")


@functools.partial(
    pl.kernel,
    mesh=_mesh,
    out_type=jax.ShapeDtypeStruct((NC, N_NODES, EMB), jnp.float32),
    scratch_types=[
        pltpu.VMEM((2, 2, CHUNK), jnp.int32),
        pltpu.VMEM((2, CHUNK, LANES), jnp.float32),
        pltpu.VMEM((2, CHUNK, EMB), jnp.float32),
        pltpu.VMEM_SHARED((N_NODES, EMB), jnp.float32),
        pltpu.SemaphoreType.DMA,
        pltpu.SemaphoreType.DMA,
        pltpu.SemaphoreType.DMA,
        pltpu.SemaphoreType.DMA,
    ],
)
def _prop(emb_hbm, idx_hbm, vals_hbm, out_hbm,
          idx_v, vals_v, buf, acc, six0, six1, sgv0, sgv1):
    cid = lax.axis_index("c")
    sid = lax.axis_index("s")
    wid = sid * NC + cid
    six = (six0, six1)
    sgv = (sgv0, sgv1)

    def fire_idx(j, s):
        pltpu.async_copy(idx_hbm.at[wid, j], idx_v.at[s], six[s])

    def wait_idx(j, s):
        pltpu.make_async_copy(idx_hbm.at[wid, j], idx_v.at[s], six[s]).wait()

    def fire_gv(j, s):
        pltpu.async_copy(emb_hbm.at[idx_v.at[s, 0]], buf.at[s], sgv[s])
        pltpu.async_copy(vals_hbm.at[wid, j], vals_v.at[s], sgv[s])

    def wait_gv(j, s):
        pltpu.make_async_copy(emb_hbm.at[idx_v.at[s, 0]], buf.at[s], sgv[s]).wait()
        pltpu.make_async_copy(vals_hbm.at[wid, j], vals_v.at[s], sgv[s]).wait()

    def scale_scatter(s):
        def _scale(i, c2):
            r0 = i * 4
            for dr in range(4):
                v = vals_v[s, r0 + dr]
                for g in range(G):
                    sl = pl.ds(g * LANES, LANES)
                    buf[s, r0 + dr, sl] = buf[s, r0 + dr, sl] * v
            return c2
        lax.fori_loop(0, CHUNK // 4, _scale, 0)
        pltpu.sync_copy(buf.at[s], acc.at[idx_v.at[s, 1]], add=True)

    def step(j, s, prefetch=True, fetch_idx=True):
        wait_gv(j, s)
        if prefetch:
            wait_idx(j + 1, 1 - s)
            fire_gv(j + 1, 1 - s)
        scale_scatter(s)
        if fetch_idx:
            fire_idx(j + 2, s)

    def _zrow(r, c):
        for g in range(G):
            buf[0, r, pl.ds(g * LANES, LANES)] = jnp.zeros((LANES,), jnp.float32)
        return c
    lax.fori_loop(0, CHUNK, _zrow, 0)
    base = sid * ROWS_PER_TILE
    for off in range(0, ROWS_PER_TILE, CHUNK):
        ln = min(CHUNK, ROWS_PER_TILE - off)
        pltpu.sync_copy(buf.at[0, pl.ds(0, ln)], acc.at[pl.ds(base + off, ln)])

    @pl.when(sid == NS - 1)
    def _():
        pltpu.sync_copy(buf.at[0, pl.ds(0, REM)], acc.at[pl.ds(REM_BASE, REM)])
    plsc.subcore_barrier()

    fire_idx(0, 0)
    wait_idx(0, 0)
    fire_gv(0, 0)
    fire_idx(1, 1)
    step(0, 0)

    def _pair(jj, c):
        step(2 * jj + 1, 1)
        step(2 * jj + 2, 0)
        return c
    lax.fori_loop(0, (CHUNKS - 4) // 2, _pair, 0)

    step(CHUNKS - 3, 1)
    step(CHUNKS - 2, 0, fetch_idx=False)
    step(CHUNKS - 1, 1, prefetch=False, fetch_idx=False)

    plsc.subcore_barrier()
    for off in range(0, ROWS_PER_TILE, CHUNK):
        ln = min(CHUNK, ROWS_PER_TILE - off)
        pltpu.sync_copy(acc.at[pl.ds(base + off, ln)],
                        out_hbm.at[cid, pl.ds(base + off, ln)])

    @pl.when(sid == NS - 1)
    def _():
        pltpu.sync_copy(acc.at[pl.ds(REM_BASE, REM)],
                        out_hbm.at[cid, pl.ds(REM_BASE, REM)])


@functools.partial(
    pl.kernel,
    mesh=_mesh,
    out_type=jax.ShapeDtypeStruct((3, BATCH, EMB), jnp.float32),
    scratch_types=[
        pltpu.VMEM((1, BATCH // NW), jnp.int32),
        pltpu.VMEM((BATCH // NW, EMB), jnp.float32),
        pltpu.SemaphoreType.DMA,
    ],
)
def _bgather(final_hbm, idx_hbm, out_hbm, idx_v, buf, sem):
    cid = lax.axis_index("c")
    sid = lax.axis_index("s")
    wid = sid * NC + cid
    bch = BATCH // NW
    base = wid * bch
    for t in range(3):
        pltpu.sync_copy(idx_hbm.at[pl.ds(t, 1), pl.ds(base, bch)], idx_v)
        if t > 0:
            for g in range(bch // LANES):
                sl = pl.ds(g * LANES, LANES)
                idx_v[0, sl] = idx_v[0, sl] + N_USERS
        pltpu.async_copy(final_hbm.at[idx_v.at[0]], buf, sem).wait()
        pltpu.sync_copy(buf, out_hbm.at[t, pl.ds(base, bch)])


def _combine_body(p_ref, s_ref, e_ref, so_ref, *, scale):
    e = p_ref[0] + p_ref[1]
    e_ref[...] = e
    so_ref[...] = (s_ref[...] + e) * scale


def _combine(p, s, scale):
    return pl.pallas_call(
        functools.partial(_combine_body, scale=scale),
        out_shape=(jax.ShapeDtypeStruct((N_NODES, EMB), jnp.float32),
                   jax.ShapeDtypeStruct((N_NODES, EMB), jnp.float32)),
    )(p, s)


_LB = 512
_LGRID = BATCH // _LB


def _loss_body(u_ref, pos_ref, neg_ref, out_ref, pn_ref, acc_ref):
    i = pl.program_id(0)

    @pl.when(i == 0)
    def _():
        pf = pos_ref[...]
        nrm = jnp.sqrt(jnp.sum(pf * pf, axis=1, keepdims=True))
        pn_ref[...] = pf / (nrm + 1e-12)

    sl = pl.ds(i * _LB, _LB)
    u = u_ref[sl, :]
    p = pos_ref[sl, :]
    ng = neg_ref[sl, :]

    pos_s = jnp.sum(u * p, axis=1)
    neg_s = jnp.sum(u * ng, axis=1)
    x = pos_s - neg_s
    log_sig = jnp.minimum(x, 0.0) - jnp.log1p(jnp.exp(-jnp.abs(x)))
    bpr_part = jnp.sum(log_sig)

    reg_part = jnp.sum(u * u) + jnp.sum(p * p) + jnp.sum(ng * ng)

    un = u / (jnp.sqrt(jnp.sum(u * u, axis=1, keepdims=True)) + 1e-12)
    pn_b = pn_ref[sl, :]
    pos_score = jnp.sum(un * pn_b, axis=1) / TAU

    logits = lax.dot_general(un, pn_ref[...], (((1,), (1,)), ((), ())),
                             preferred_element_type=jnp.float32) / TAU
    m = jnp.max(logits, axis=1)
    ttl = m + jnp.log(jnp.sum(jnp.exp(logits - m[:, None]), axis=1))
    na_part = jnp.sum(ttl - pos_score)

    @pl.when(i == 0)
    def _():
        acc_ref[0] = bpr_part
        acc_ref[1] = reg_part
        acc_ref[2] = na_part

    @pl.when(i > 0)
    def _():
        acc_ref[0] += bpr_part
        acc_ref[1] += reg_part
        acc_ref[2] += na_part

    @pl.when(i == _LGRID - 1)
    def _():
        bpr = -acc_ref[0] / BATCH
        reg = 0.5 * acc_ref[1] / BATCH
        na = acc_ref[2] / BATCH
        out_ref[0] = bpr + REG_LAMBDA * reg + SSL_LAMBDA * na


def _loss(u, pos, neg):
    full = pl.BlockSpec((BATCH, EMB), lambda i: (0, 0))
    return pl.pallas_call(
        _loss_body,
        grid=(_LGRID,),
        in_specs=[full, full, full],
        out_specs=pl.BlockSpec(memory_space=pltpu.SMEM),
        out_shape=jax.ShapeDtypeStruct((1,), jnp.float32),
        scratch_shapes=[
            pltpu.VMEM((BATCH, EMB), jnp.float32),
            pltpu.SMEM((3,), jnp.float32),
        ],
    )(u, pos, neg)


def kernel(user, pos_item, neg_item, adj_indices, adj_values,
           user_emb_w, item_emb_w):
    emb0 = jnp.concatenate([user_emb_w, item_emb_w], axis=0)

    rows = adj_indices[0].astype(jnp.int32)
    cols = adj_indices[1].astype(jnp.int32)
    vals = adj_values.astype(jnp.float32)
    pad = E_PAD - E
    rows_p = jnp.pad(rows, (0, pad)).reshape(NW, CHUNKS, CHUNK)
    cols_p = jnp.pad(cols, (0, pad)).reshape(NW, CHUNKS, CHUNK)
    idx_pack = jnp.stack([cols_p, rows_p], axis=2)
    vals16 = jnp.broadcast_to(jnp.pad(vals, (0, pad))[:, None],
                              (E_PAD, LANES)).reshape(NW, CHUNKS, CHUNK, LANES)

    e = emb0
    s = emb0
    for layer in range(3):
        parts = _prop(e, idx_pack, vals16)
        scale = 0.25 if layer == 2 else 1.0
        e, s = _combine(parts, s, scale)
    final = s

    idx = jnp.stack([user.astype(jnp.int32),
                     pos_item.astype(jnp.int32),
                     neg_item.astype(jnp.int32)])
    upn = _bgather(final, idx)
    total = _loss(upn[0], upn[1], upn[2])
    return total.reshape(())


# speedup vs baseline: 1.2558x; 1.0735x over previous
import functools

import jax
import jax.numpy as jnp
from jax import lax
from jax.experimental import pallas as pl
from jax.experimental.pallas import tpu as pltpu
from jax.experimental.pallas import tpu_sc as plsc

N_USERS = 5000
N_ITEMS = 5000
N_NODES = N_USERS + N_ITEMS
EMB = 128
E = 320000
BATCH = 4096
TAU = 0.2
SSL_LAMBDA = 0.1
REG_LAMBDA = 1e-4

NC = 2
NS = 16
NW = NC * NS
LANES = 16
G = EMB // LANES

CHUNK = 64
CHUNKS = 158
E_PAD = NW * CHUNKS * CHUNK
ROWS_PER_TILE = 624
REM_BASE = NS * ROWS_PER_TILE
REM = N_NODES - REM_BASE

_mesh = plsc.VectorSubcoreMesh(core_axis_name="c", subcore_axis_name="s")


@functools.partial(
    pl.kernel,
    mesh=_mesh,
    out_type=jax.ShapeDtypeStruct((NC, N_NODES, EMB), jnp.float32),
    scratch_types=[
        pltpu.VMEM((2, 2, CHUNK), jnp.int32),
        pltpu.VMEM((2, CHUNK, LANES), jnp.float32),
        pltpu.VMEM((2, CHUNK, EMB), jnp.float32),
        pltpu.VMEM_SHARED((N_NODES, EMB), jnp.float32),
        pltpu.SemaphoreType.DMA,
        pltpu.SemaphoreType.DMA,
        pltpu.SemaphoreType.DMA,
        pltpu.SemaphoreType.DMA,
    ],
)
def _prop(emb_hbm, idx_hbm, vals_hbm, out_hbm,
          idx_v, vals_v, buf, acc, six0, six1, sgv0, sgv1):
    cid = lax.axis_index("c")
    sid = lax.axis_index("s")
    wid = sid * NC + cid
    six = (six0, six1)
    sgv = (sgv0, sgv1)

    def fire_idx(j, s):
        pltpu.async_copy(idx_hbm.at[wid, j], idx_v.at[s], six[s])

    def wait_idx(j, s):
        pltpu.make_async_copy(idx_hbm.at[wid, j], idx_v.at[s], six[s]).wait()

    def fire_gv(j, s):
        pltpu.async_copy(emb_hbm.at[idx_v.at[s, 0]], buf.at[s], sgv[s])
        pltpu.async_copy(vals_hbm.at[wid, j], vals_v.at[s], sgv[s])

    def wait_gv(j, s):
        pltpu.make_async_copy(emb_hbm.at[idx_v.at[s, 0]], buf.at[s], sgv[s]).wait()
        pltpu.make_async_copy(vals_hbm.at[wid, j], vals_v.at[s], sgv[s]).wait()

    def scale_scatter(s):
        def _scale(i, c2):
            r0 = i * 4
            for dr in range(4):
                v = vals_v[s, r0 + dr]
                for g in range(G):
                    sl = pl.ds(g * LANES, LANES)
                    buf[s, r0 + dr, sl] = buf[s, r0 + dr, sl] * v
            return c2
        lax.fori_loop(0, CHUNK // 4, _scale, 0)
        pltpu.sync_copy(buf.at[s], acc.at[idx_v.at[s, 1]], add=True)

    def step(j, s, prefetch=True, fetch_idx=True):
        wait_gv(j, s)
        if prefetch:
            wait_idx(j + 1, 1 - s)
            fire_gv(j + 1, 1 - s)
        scale_scatter(s)
        if fetch_idx:
            fire_idx(j + 2, s)

    def _zrow(r, c):
        for g in range(G):
            buf[0, r, pl.ds(g * LANES, LANES)] = jnp.zeros((LANES,), jnp.float32)
        return c
    lax.fori_loop(0, CHUNK, _zrow, 0)
    base = sid * ROWS_PER_TILE
    for off in range(0, ROWS_PER_TILE, CHUNK):
        ln = min(CHUNK, ROWS_PER_TILE - off)
        pltpu.sync_copy(buf.at[0, pl.ds(0, ln)], acc.at[pl.ds(base + off, ln)])

    @pl.when(sid == NS - 1)
    def _():
        pltpu.sync_copy(buf.at[0, pl.ds(0, REM)], acc.at[pl.ds(REM_BASE, REM)])
    plsc.subcore_barrier()

    fire_idx(0, 0)
    wait_idx(0, 0)
    fire_gv(0, 0)
    fire_idx(1, 1)
    step(0, 0)

    def _pair(jj, c):
        step(2 * jj + 1, 1)
        step(2 * jj + 2, 0)
        return c
    lax.fori_loop(0, (CHUNKS - 4) // 2, _pair, 0)

    step(CHUNKS - 3, 1)
    step(CHUNKS - 2, 0, fetch_idx=False)
    step(CHUNKS - 1, 1, prefetch=False, fetch_idx=False)

    plsc.subcore_barrier()
    for off in range(0, ROWS_PER_TILE, CHUNK):
        ln = min(CHUNK, ROWS_PER_TILE - off)
        pltpu.sync_copy(acc.at[pl.ds(base + off, ln)],
                        out_hbm.at[cid, pl.ds(base + off, ln)])

    @pl.when(sid == NS - 1)
    def _():
        pltpu.sync_copy(acc.at[pl.ds(REM_BASE, REM)],
                        out_hbm.at[cid, pl.ds(REM_BASE, REM)])


@functools.partial(
    pl.kernel,
    mesh=_mesh,
    out_type=jax.ShapeDtypeStruct((3, BATCH, EMB), jnp.float32),
    scratch_types=[
        pltpu.VMEM((1, BATCH // NW), jnp.int32),
        pltpu.VMEM((BATCH // NW, EMB), jnp.float32),
        pltpu.SemaphoreType.DMA,
    ],
)
def _bgather(final_hbm, idx_hbm, out_hbm, idx_v, buf, sem):
    cid = lax.axis_index("c")
    sid = lax.axis_index("s")
    wid = sid * NC + cid
    bch = BATCH // NW
    base = wid * bch
    for t in range(3):
        pltpu.sync_copy(idx_hbm.at[pl.ds(t, 1), pl.ds(base, bch)], idx_v)
        if t > 0:
            for g in range(bch // LANES):
                sl = pl.ds(g * LANES, LANES)
                idx_v[0, sl] = idx_v[0, sl] + N_USERS
        pltpu.async_copy(final_hbm.at[idx_v.at[0]], buf, sem).wait()
        pltpu.sync_copy(buf, out_hbm.at[t, pl.ds(base, bch)])


def _combine_body(p_ref, s_ref, e_ref, so_ref, *, scale):
    e = p_ref[0] + p_ref[1]
    e_ref[...] = e
    so_ref[...] = (s_ref[...] + e) * scale


def _combine(p, s, scale):
    return pl.pallas_call(
        functools.partial(_combine_body, scale=scale),
        out_shape=(jax.ShapeDtypeStruct((N_NODES, EMB), jnp.float32),
                   jax.ShapeDtypeStruct((N_NODES, EMB), jnp.float32)),
    )(p, s)


_LB = 512
_LGRID = BATCH // _LB


def _loss_body(u_ref, pos_ref, neg_ref, out_ref, pn_ref, acc_ref):
    i = pl.program_id(0)

    @pl.when(i == 0)
    def _():
        pf = pos_ref[...]
        nrm = jnp.sqrt(jnp.sum(pf * pf, axis=1, keepdims=True))
        pn_ref[...] = pf / (nrm + 1e-12)

    sl = pl.ds(i * _LB, _LB)
    u = u_ref[sl, :]
    p = pos_ref[sl, :]
    ng = neg_ref[sl, :]

    pos_s = jnp.sum(u * p, axis=1)
    neg_s = jnp.sum(u * ng, axis=1)
    x = pos_s - neg_s
    log_sig = jnp.minimum(x, 0.0) - jnp.log1p(jnp.exp(-jnp.abs(x)))
    bpr_part = jnp.sum(log_sig)

    reg_part = jnp.sum(u * u) + jnp.sum(p * p) + jnp.sum(ng * ng)

    un = u / (jnp.sqrt(jnp.sum(u * u, axis=1, keepdims=True)) + 1e-12)
    pn_b = pn_ref[sl, :]
    pos_score = jnp.sum(un * pn_b, axis=1) / TAU

    logits = lax.dot_general(un, pn_ref[...], (((1,), (1,)), ((), ())),
                             preferred_element_type=jnp.float32) / TAU
    m = jnp.max(logits, axis=1)
    ttl = m + jnp.log(jnp.sum(jnp.exp(logits - m[:, None]), axis=1))
    na_part = jnp.sum(ttl - pos_score)

    @pl.when(i == 0)
    def _():
        acc_ref[0] = bpr_part
        acc_ref[1] = reg_part
        acc_ref[2] = na_part

    @pl.when(i > 0)
    def _():
        acc_ref[0] += bpr_part
        acc_ref[1] += reg_part
        acc_ref[2] += na_part

    @pl.when(i == _LGRID - 1)
    def _():
        bpr = -acc_ref[0] / BATCH
        reg = 0.5 * acc_ref[1] / BATCH
        na = acc_ref[2] / BATCH
        out_ref[0] = bpr + REG_LAMBDA * reg + SSL_LAMBDA * na


def _loss(u, pos, neg):
    full = pl.BlockSpec((BATCH, EMB), lambda i: (0, 0))
    return pl.pallas_call(
        _loss_body,
        grid=(_LGRID,),
        in_specs=[full, full, full],
        out_specs=pl.BlockSpec(memory_space=pltpu.SMEM),
        out_shape=jax.ShapeDtypeStruct((1,), jnp.float32),
        scratch_shapes=[
            pltpu.VMEM((BATCH, EMB), jnp.float32),
            pltpu.SMEM((3,), jnp.float32),
        ],
    )(u, pos, neg)


def kernel(user, pos_item, neg_item, adj_indices, adj_values,
           user_emb_w, item_emb_w):
    emb0 = jnp.concatenate([user_emb_w, item_emb_w], axis=0)

    rows = adj_indices[0].astype(jnp.int32)
    cols = adj_indices[1].astype(jnp.int32)
    vals = adj_values.astype(jnp.float32)
    pad = E_PAD - E
    rows_p = jnp.pad(rows, (0, pad)).reshape(NW, CHUNKS, CHUNK)
    cols_p = jnp.pad(cols, (0, pad)).reshape(NW, CHUNKS, CHUNK)
    idx_pack = jnp.stack([cols_p, rows_p], axis=2)
    vals16 = jnp.broadcast_to(jnp.pad(vals, (0, pad))[:, None],
                              (E_PAD, LANES)).reshape(NW, CHUNKS, CHUNK, LANES)

    e = emb0
    s = emb0
    for layer in range(3):
        parts = _prop(e, idx_pack, vals16)
        scale = 0.25 if layer == 2 else 1.0
        e, s = _combine(parts, s, scale)
    final = s

    idx = jnp.stack([user.astype(jnp.int32),
                     pos_item.astype(jnp.int32),
                     neg_item.astype(jnp.int32)])
    upn = _bgather(final, idx)
    total = _loss(upn[0], upn[1], upn[2])
    return total.reshape(())


# speedup vs baseline: 1.3665x; 1.0882x over previous
import functools

import jax
import jax.numpy as jnp
from jax import lax
from jax.experimental import pallas as pl
from jax.experimental.pallas import tpu as pltpu
from jax.experimental.pallas import tpu_sc as plsc

N_USERS = 5000
N_ITEMS = 5000
N_NODES = N_USERS + N_ITEMS
EMB = 128
E = 320000
BATCH = 4096
TAU = 0.2
SSL_LAMBDA = 0.1
REG_LAMBDA = 1e-4

NC = 2
NS = 16
NW = NC * NS
LANES = 16
G = EMB // LANES

CHUNK = 64
CHUNKS = 158
E_PAD = NW * CHUNKS * CHUNK
ROWS_PER_TILE = 624
REM_BASE = NS * ROWS_PER_TILE
REM = N_NODES - REM_BASE

_mesh = plsc.VectorSubcoreMesh(core_axis_name="c", subcore_axis_name="s")


@functools.partial(
    pl.kernel,
    mesh=_mesh,
    out_type=jax.ShapeDtypeStruct((NC, N_NODES, EMB), jnp.float32),
    scratch_types=[
        pltpu.VMEM((2, 2, CHUNK), jnp.int32),
        pltpu.VMEM((2, 1, CHUNK), jnp.float32),
        pltpu.VMEM((2, CHUNK, EMB), jnp.float32),
        pltpu.VMEM_SHARED((N_NODES, EMB), jnp.float32),
        pltpu.SemaphoreType.DMA,
        pltpu.SemaphoreType.DMA,
        pltpu.SemaphoreType.DMA,
        pltpu.SemaphoreType.DMA,
    ],
)
def _prop(emb_hbm, idx_hbm, vals_hbm, out_hbm,
          idx_v, vals_v, buf, acc, six0, six1, sgv0, sgv1):
    cid = lax.axis_index("c")
    sid = lax.axis_index("s")
    wid = sid * NC + cid
    six = (six0, six1)
    sgv = (sgv0, sgv1)

    def fire_idx(j, s):
        pltpu.async_copy(idx_hbm.at[wid, j], idx_v.at[s], six[s])

    def wait_idx(j, s):
        pltpu.make_async_copy(idx_hbm.at[wid, j], idx_v.at[s], six[s]).wait()

    def fire_gv(j, s):
        pltpu.async_copy(emb_hbm.at[idx_v.at[s, 0]], buf.at[s], sgv[s])
        pltpu.async_copy(vals_hbm.at[wid, pl.ds(j, 1)], vals_v.at[s], sgv[s])

    def wait_gv(j, s):
        pltpu.make_async_copy(emb_hbm.at[idx_v.at[s, 0]], buf.at[s], sgv[s]).wait()
        pltpu.make_async_copy(vals_hbm.at[wid, pl.ds(j, 1)], vals_v.at[s],
                              sgv[s]).wait()

    def scale_scatter(s):
        def _scale(i, c2):
            r0 = i * LANES
            vv = vals_v[s, 0, pl.ds(r0, LANES)]
            for k in range(LANES):
                v = vv[k]
                for g in range(G):
                    sl = pl.ds(g * LANES, LANES)
                    buf[s, r0 + k, sl] = buf[s, r0 + k, sl] * v
            return c2
        lax.fori_loop(0, CHUNK // LANES, _scale, 0)
        pltpu.sync_copy(buf.at[s], acc.at[idx_v.at[s, 1]], add=True)

    def step(j, s, prefetch=True, fetch_idx=True):
        wait_gv(j, s)
        if prefetch:
            wait_idx(j + 1, 1 - s)
            fire_gv(j + 1, 1 - s)
        scale_scatter(s)
        if fetch_idx:
            fire_idx(j + 2, s)

    def _zrow(r, c):
        for g in range(G):
            buf[0, r, pl.ds(g * LANES, LANES)] = jnp.zeros((LANES,), jnp.float32)
        return c
    lax.fori_loop(0, CHUNK, _zrow, 0)
    base = sid * ROWS_PER_TILE
    for off in range(0, ROWS_PER_TILE, CHUNK):
        ln = min(CHUNK, ROWS_PER_TILE - off)
        pltpu.sync_copy(buf.at[0, pl.ds(0, ln)], acc.at[pl.ds(base + off, ln)])

    @pl.when(sid == NS - 1)
    def _():
        pltpu.sync_copy(buf.at[0, pl.ds(0, REM)], acc.at[pl.ds(REM_BASE, REM)])
    plsc.subcore_barrier()

    fire_idx(0, 0)
    wait_idx(0, 0)
    fire_gv(0, 0)
    fire_idx(1, 1)
    step(0, 0)

    def _pair(jj, c):
        step(2 * jj + 1, 1)
        step(2 * jj + 2, 0)
        return c
    lax.fori_loop(0, (CHUNKS - 4) // 2, _pair, 0)

    step(CHUNKS - 3, 1)
    step(CHUNKS - 2, 0, fetch_idx=False)
    step(CHUNKS - 1, 1, prefetch=False, fetch_idx=False)

    plsc.subcore_barrier()
    for off in range(0, ROWS_PER_TILE, CHUNK):
        ln = min(CHUNK, ROWS_PER_TILE - off)
        pltpu.sync_copy(acc.at[pl.ds(base + off, ln)],
                        out_hbm.at[cid, pl.ds(base + off, ln)])

    @pl.when(sid == NS - 1)
    def _():
        pltpu.sync_copy(acc.at[pl.ds(REM_BASE, REM)],
                        out_hbm.at[cid, pl.ds(REM_BASE, REM)])


@functools.partial(
    pl.kernel,
    mesh=_mesh,
    out_type=jax.ShapeDtypeStruct((3, BATCH, EMB), jnp.float32),
    scratch_types=[
        pltpu.VMEM((1, BATCH // NW), jnp.int32),
        pltpu.VMEM((BATCH // NW, EMB), jnp.float32),
        pltpu.SemaphoreType.DMA,
    ],
)
def _bgather(final_hbm, idx_hbm, out_hbm, idx_v, buf, sem):
    cid = lax.axis_index("c")
    sid = lax.axis_index("s")
    wid = sid * NC + cid
    bch = BATCH // NW
    base = wid * bch
    for t in range(3):
        pltpu.sync_copy(idx_hbm.at[pl.ds(t, 1), pl.ds(base, bch)], idx_v)
        if t > 0:
            for g in range(bch // LANES):
                sl = pl.ds(g * LANES, LANES)
                idx_v[0, sl] = idx_v[0, sl] + N_USERS
        pltpu.async_copy(final_hbm.at[idx_v.at[0]], buf, sem).wait()
        pltpu.sync_copy(buf, out_hbm.at[t, pl.ds(base, bch)])


def _combine_body(p_ref, s_ref, e_ref, so_ref, *, scale):
    e = p_ref[0] + p_ref[1]
    e_ref[...] = e
    so_ref[...] = (s_ref[...] + e) * scale


def _combine(p, s, scale):
    return pl.pallas_call(
        functools.partial(_combine_body, scale=scale),
        out_shape=(jax.ShapeDtypeStruct((N_NODES, EMB), jnp.float32),
                   jax.ShapeDtypeStruct((N_NODES, EMB), jnp.float32)),
    )(p, s)


_LB = 512
_LGRID = BATCH // _LB


def _loss_body(u_ref, pos_ref, neg_ref, out_ref, pn_ref, acc_ref):
    i = pl.program_id(0)

    @pl.when(i == 0)
    def _():
        pf = pos_ref[...]
        nrm = jnp.sqrt(jnp.sum(pf * pf, axis=1, keepdims=True))
        pn_ref[...] = pf / (nrm + 1e-12)

    sl = pl.ds(i * _LB, _LB)
    u = u_ref[sl, :]
    p = pos_ref[sl, :]
    ng = neg_ref[sl, :]

    pos_s = jnp.sum(u * p, axis=1)
    neg_s = jnp.sum(u * ng, axis=1)
    x = pos_s - neg_s
    log_sig = jnp.minimum(x, 0.0) - jnp.log1p(jnp.exp(-jnp.abs(x)))
    bpr_part = jnp.sum(log_sig)

    reg_part = jnp.sum(u * u) + jnp.sum(p * p) + jnp.sum(ng * ng)

    un = u / (jnp.sqrt(jnp.sum(u * u, axis=1, keepdims=True)) + 1e-12)
    pn_b = pn_ref[sl, :]
    pos_score = jnp.sum(un * pn_b, axis=1) / TAU

    logits = lax.dot_general(un, pn_ref[...], (((1,), (1,)), ((), ())),
                             preferred_element_type=jnp.float32) / TAU
    m = jnp.max(logits, axis=1)
    ttl = m + jnp.log(jnp.sum(jnp.exp(logits - m[:, None]), axis=1))
    na_part = jnp.sum(ttl - pos_score)

    @pl.when(i == 0)
    def _():
        acc_ref[0] = bpr_part
        acc_ref[1] = reg_part
        acc_ref[2] = na_part

    @pl.when(i > 0)
    def _():
        acc_ref[0] += bpr_part
        acc_ref[1] += reg_part
        acc_ref[2] += na_part

    @pl.when(i == _LGRID - 1)
    def _():
        bpr = -acc_ref[0] / BATCH
        reg = 0.5 * acc_ref[1] / BATCH
        na = acc_ref[2] / BATCH
        out_ref[0] = bpr + REG_LAMBDA * reg + SSL_LAMBDA * na


def _loss(u, pos, neg):
    full = pl.BlockSpec((BATCH, EMB), lambda i: (0, 0))
    return pl.pallas_call(
        _loss_body,
        grid=(_LGRID,),
        in_specs=[full, full, full],
        out_specs=pl.BlockSpec(memory_space=pltpu.SMEM),
        out_shape=jax.ShapeDtypeStruct((1,), jnp.float32),
        scratch_shapes=[
            pltpu.VMEM((BATCH, EMB), jnp.float32),
            pltpu.SMEM((3,), jnp.float32),
        ],
    )(u, pos, neg)


def kernel(user, pos_item, neg_item, adj_indices, adj_values,
           user_emb_w, item_emb_w):
    emb0 = jnp.concatenate([user_emb_w, item_emb_w], axis=0)

    rows = adj_indices[0].astype(jnp.int32)
    cols = adj_indices[1].astype(jnp.int32)
    vals = adj_values.astype(jnp.float32)
    pad = E_PAD - E
    rows_p = jnp.pad(rows, (0, pad)).reshape(NW, CHUNKS, CHUNK)
    cols_p = jnp.pad(cols, (0, pad)).reshape(NW, CHUNKS, CHUNK)
    idx_pack = jnp.stack([cols_p, rows_p], axis=2)
    vals_p = jnp.pad(vals, (0, pad)).reshape(NW, CHUNKS, CHUNK)

    e = emb0
    s = emb0
    for layer in range(3):
        parts = _prop(e, idx_pack, vals_p)
        scale = 0.25 if layer == 2 else 1.0
        e, s = _combine(parts, s, scale)
    final = s

    idx = jnp.stack([user.astype(jnp.int32),
                     pos_item.astype(jnp.int32),
                     neg_item.astype(jnp.int32)])
    upn = _bgather(final, idx)
    total = _loss(upn[0], upn[1], upn[2])
    return total.reshape(())
